# baseline (speedup 1.0000x reference)
import functools

import jax
import jax.numpy as jnp
from jax import lax
from jax.experimental import pallas as pl
from jax.experimental.pallas import tpu as pltpu
from jax.experimental.pallas import tpu_sc as plsc

_VOCAB_BLK = 32768


def _proj_body(w_ref, b_ref, emb_ref, out_ref):
    w = w_ref[...]
    h = emb_ref[...]
    y = jax.lax.dot_general(
        w, h, (((1,), (1,)), ((), ())),
        preferred_element_type=jnp.float32)
    out_ref[0] = jax.nn.sigmoid(y + b_ref[0, 0])


def _project(emb, W_fc, b_fc):
    V = emb.shape[0]
    nb = pl.cdiv(V, _VOCAB_BLK)
    p2d = pl.pallas_call(
        _proj_body,
        grid=(nb,),
        in_specs=[
            pl.BlockSpec((1, emb.shape[1]), lambda i: (0, 0)),
            pl.BlockSpec((1, 1), lambda i: (0, 0)),
            pl.BlockSpec((_VOCAB_BLK, emb.shape[1]), lambda i: (i, 0)),
        ],
        out_specs=pl.BlockSpec((1, 1, _VOCAB_BLK), lambda i: (i, 0, 0)),
        out_shape=jax.ShapeDtypeStruct((nb, 1, _VOCAB_BLK), jnp.float32),
    )(W_fc, b_fc.reshape(1, 1), emb)
    return p2d


def _gather(p_flat, idx):
    info = plsc.get_sparse_core_info()
    NC, NS = info.num_cores, info.num_subcores
    NW = NC * NS
    B = idx.shape[0]
    b_per_w = B // NW
    mesh = plsc.VectorSubcoreMesh(core_axis_name="c", subcore_axis_name="s")

    @functools.partial(
        pl.kernel,
        mesh=mesh,
        out_type=jax.ShapeDtypeStruct((B,), jnp.float32),
        scratch_types=[
            pltpu.VMEM((b_per_w,), jnp.int32),
            pltpu.VMEM((b_per_w,), jnp.float32),
            pltpu.SemaphoreType.DMA,
        ],
    )
    def gk(p_hbm, idx_hbm, out_hbm, idx_v, vals_v, sem):
        wid = lax.axis_index("s") * NC + lax.axis_index("c")
        base = wid * b_per_w
        pltpu.sync_copy(idx_hbm.at[pl.ds(base, b_per_w)], idx_v)
        pltpu.async_copy(p_hbm.at[idx_v], vals_v, sem).wait()
        pltpu.sync_copy(vals_v, out_hbm.at[pl.ds(base, b_per_w)])

    return gk(p_flat, idx)


def kernel(x, emb, W_fc, b_fc):
    e2 = emb.reshape(emb.shape[0] // 2, 128)
    w2 = jnp.concatenate([W_fc, W_fc], axis=1)
    V2 = e2.shape[0]
    nb = pl.cdiv(V2, _VOCAB_BLK)
    return pl.pallas_call(
        _proj_body,
        grid=(nb,),
        in_specs=[
            pl.BlockSpec((1, 128), lambda i: (0, 0)),
            pl.BlockSpec((1, 1), lambda i: (0, 0)),
            pl.BlockSpec((_VOCAB_BLK, 128), lambda i: (i, 0)),
        ],
        out_specs=pl.BlockSpec((1, 1, _VOCAB_BLK), lambda i: (i, 0, 0)),
        out_shape=jax.ShapeDtypeStruct((nb, 1, _VOCAB_BLK), jnp.float32),
    )(w2, b_fc.reshape(1, 1), e2)


# speedup vs baseline: 5.0082x; 5.0082x over previous
import functools

import jax
import jax.numpy as jnp
from jax import lax
from jax.experimental import pallas as pl
from jax.experimental.pallas import tpu as pltpu
from jax.experimental.pallas import tpu_sc as plsc

_VOCAB_BLK = 32768


def _proj_body(w_ref, b_ref, embt_ref, out_ref):
    w = w_ref[...]
    ht = embt_ref[...]
    y = jax.lax.dot_general(
        w, ht, (((1,), (0,)), ((), ())),
        preferred_element_type=jnp.float32)
    out_ref[0] = jax.nn.sigmoid(y + b_ref[0, 0])


def _project(embt, W_fc, b_fc):
    D, V = embt.shape
    nb = pl.cdiv(V, _VOCAB_BLK)
    p2d = pl.pallas_call(
        _proj_body,
        grid=(nb,),
        in_specs=[
            pl.BlockSpec((1, D), lambda i: (0, 0)),
            pl.BlockSpec((1, 1), lambda i: (0, 0)),
            pl.BlockSpec((D, _VOCAB_BLK), lambda i: (0, i)),
        ],
        out_specs=pl.BlockSpec((1, 1, _VOCAB_BLK), lambda i: (i, 0, 0)),
        out_shape=jax.ShapeDtypeStruct((nb, 1, _VOCAB_BLK), jnp.float32),
    )(W_fc, b_fc.reshape(1, 1), embt)
    return p2d


def _gather(p_flat, idx):
    info = plsc.get_sparse_core_info()
    NC, NS = info.num_cores, info.num_subcores
    NW = NC * NS
    B = idx.shape[0]
    b_per_w = B // NW
    mesh = plsc.VectorSubcoreMesh(core_axis_name="c", subcore_axis_name="s")

    @functools.partial(
        pl.kernel,
        mesh=mesh,
        out_type=jax.ShapeDtypeStruct((B,), jnp.float32),
        scratch_types=[
            pltpu.VMEM((b_per_w,), jnp.int32),
            pltpu.VMEM((b_per_w,), jnp.float32),
            pltpu.SemaphoreType.DMA,
        ],
    )
    def gk(p_hbm, idx_hbm, out_hbm, idx_v, vals_v, sem):
        wid = lax.axis_index("s") * NC + lax.axis_index("c")
        base = wid * b_per_w
        pltpu.sync_copy(idx_hbm.at[pl.ds(base, b_per_w)], idx_v)
        pltpu.async_copy(p_hbm.at[idx_v], vals_v, sem).wait()
        pltpu.sync_copy(vals_v, out_hbm.at[pl.ds(base, b_per_w)])

    return gk(p_flat, idx)


def kernel(x, emb, W_fc, b_fc):
    Bt, S = x.shape
    idx_t = x.T.reshape(-1).astype(jnp.int32)
    p = _project(emb.T, W_fc, b_fc)
    out_t = _gather(p.reshape(-1), idx_t)
    return out_t.reshape(S, Bt).T.reshape(Bt, S, 1)
